# baseline (speedup 1.0000x reference)
import functools

import jax
import jax.numpy as jnp
from jax import lax
from jax.experimental import pallas as pl
from jax.experimental.pallas import tpu as pltpu
from jax.experimental.pallas import tpu_sc as plsc

N = 10000
E = 320000
IN_CH = 128
D = 64
NP = 10240
NCORES = 2
NSUB = 16
NTILES = NCORES * NSUB
CHUNK = 128
CPT = 80
EP = NTILES * CPT * CHUNK
STRIPE = NP // NSUB
DEGW = 16
RB = 1024

_mesh = plsc.VectorSubcoreMesh(core_axis_name="c", subcore_axis_name="s")
_SC_PARAMS = pltpu.CompilerParams(use_tc_tiling_on_sc=False)


@functools.partial(
    pl.kernel,
    out_type=jax.ShapeDtypeStruct((NCORES, NP, DEGW), jnp.float32),
    mesh=_mesh,
    scratch_types=[
        pltpu.VMEM((CPT, CHUNK), jnp.int32),
        pltpu.VMEM((CHUNK, DEGW), jnp.float32),
        pltpu.VMEM_SHARED((NP, DEGW), jnp.float32),
    ],
    compiler_params=_SC_PARAMS,
)
def _deg_kernel(dst_hbm, ones_hbm, zeros_hbm, out_hbm, didx, ones_v, acc):
    cid = lax.axis_index("c")
    sid = lax.axis_index("s")
    wid = sid * NCORES + cid
    pltpu.sync_copy(zeros_hbm.at[pl.ds(sid * STRIPE, STRIPE)],
                    acc.at[pl.ds(sid * STRIPE, STRIPE)])
    pltpu.sync_copy(dst_hbm.at[wid], didx)
    pltpu.sync_copy(ones_hbm, ones_v)
    plsc.subcore_barrier()

    @pl.loop(0, CPT)
    def _(j):
        pltpu.sync_copy(ones_v, acc.at[didx.at[j]], add=True)

    plsc.subcore_barrier()
    pltpu.sync_copy(acc.at[pl.ds(sid * STRIPE, STRIPE)],
                    out_hbm.at[cid, pl.ds(sid * STRIPE, STRIPE)])


@functools.partial(
    pl.kernel,
    out_type=jax.ShapeDtypeStruct((NCORES, NP, D), jnp.float32),
    mesh=_mesh,
    scratch_types=[
        pltpu.VMEM((CPT, CHUNK), jnp.int32),
        pltpu.VMEM((CPT, CHUNK), jnp.int32),
        pltpu.VMEM((CHUNK, D), jnp.float32),
        pltpu.VMEM_SHARED((NP, D), jnp.float32),
    ],
    compiler_params=_SC_PARAMS,
)
def _hop_kernel(t_hbm, src_hbm, dst_hbm, zeros_hbm, out_hbm,
                sidx, didx, rows_a, acc):
    cid = lax.axis_index("c")
    sid = lax.axis_index("s")
    wid = sid * NCORES + cid
    pltpu.sync_copy(zeros_hbm.at[pl.ds(sid * STRIPE, STRIPE)],
                    acc.at[pl.ds(sid * STRIPE, STRIPE)])
    pltpu.sync_copy(src_hbm.at[wid], sidx)
    pltpu.sync_copy(dst_hbm.at[wid], didx)
    plsc.subcore_barrier()

    @pl.loop(0, CPT)
    def _(j):
        pltpu.sync_copy(t_hbm.at[sidx.at[j]], rows_a)
        pltpu.sync_copy(rows_a, acc.at[didx.at[j]], add=True)

    plsc.subcore_barrier()
    pltpu.sync_copy(acc.at[pl.ds(sid * STRIPE, STRIPE)],
                    out_hbm.at[cid, pl.ds(sid * STRIPE, STRIPE)])


def _mm_body(x_ref, w_ref, o_ref):
    o_ref[...] = lax.dot_general(
        x_ref[...], w_ref[...], (((1,), (1,)), ((), ())),
        preferred_element_type=jnp.float32,
        precision=lax.Precision.HIGHEST)


_matmul = pl.pallas_call(
    _mm_body,
    grid=(NP // RB,),
    in_specs=[pl.BlockSpec((RB, IN_CH), lambda i: (i, 0)),
              pl.BlockSpec((D, IN_CH), lambda i: (0, 0))],
    out_specs=pl.BlockSpec((RB, D), lambda i: (i, 0)),
    out_shape=jax.ShapeDtypeStruct((NP, D), jnp.float32),
)


def _deg_of(dp_ref):
    return dp_ref[0, :, 0:1] + dp_ref[1, :, 0:1] + 1.0


def _scale_body(h_ref, dp_ref, o_ref):
    o_ref[...] = h_ref[...] * lax.rsqrt(_deg_of(dp_ref))


def _mid_body(q_ref, t_ref, dp_ref, o_ref):
    o_ref[...] = (q_ref[0] + q_ref[1] + t_ref[...]) / _deg_of(dp_ref)


def _fin_body(r_ref, t_ref, dp_ref, b_ref, o_ref):
    o_ref[...] = ((r_ref[0] + r_ref[1] + t_ref[...])
                  * lax.rsqrt(_deg_of(dp_ref)) + b_ref[...])


_T_SPEC = pl.BlockSpec((RB, D), lambda i: (i, 0))
_P_SPEC = pl.BlockSpec((NCORES, RB, D), lambda i: (0, i, 0))
_DP_SPEC = pl.BlockSpec((NCORES, RB, DEGW), lambda i: (0, i, 0))
_OUT_T = jax.ShapeDtypeStruct((NP, D), jnp.float32)

_scale = pl.pallas_call(
    _scale_body, grid=(NP // RB,),
    in_specs=[_T_SPEC, _DP_SPEC], out_specs=_T_SPEC, out_shape=_OUT_T)

_mid = pl.pallas_call(
    _mid_body, grid=(NP // RB,),
    in_specs=[_P_SPEC, _T_SPEC, _DP_SPEC], out_specs=_T_SPEC, out_shape=_OUT_T)

_fin = pl.pallas_call(
    _fin_body, grid=(NP // RB,),
    in_specs=[_P_SPEC, _T_SPEC, _DP_SPEC,
              pl.BlockSpec((1, D), lambda i: (0, 0))],
    out_specs=_T_SPEC, out_shape=_OUT_T)


def kernel(x, edge_index, W, b):
    src = edge_index[0].astype(jnp.int32)
    dst = edge_index[1].astype(jnp.int32)
    pad = EP - E
    pad_dst = N + jnp.arange(pad, dtype=jnp.int32) % (NP - N)
    src3 = jnp.concatenate(
        [src, jnp.zeros((pad,), jnp.int32)]).reshape(NTILES, CPT, CHUNK)
    dst3 = jnp.concatenate([dst, pad_dst]).reshape(NTILES, CPT, CHUNK)
    xp = jnp.pad(x, ((0, NP - N), (0, 0)))
    zeros_d = jnp.zeros((NP, D), jnp.float32)
    zeros_g = jnp.zeros((NP, DEGW), jnp.float32)
    ones_g = jnp.ones((CHUNK, DEGW), jnp.float32)

    h0 = _matmul(xp, W)
    dp = _deg_kernel(dst3, ones_g, zeros_g)
    t0 = _scale(h0, dp)
    q = _hop_kernel(t0, src3, dst3, zeros_d)
    t1 = _mid(q, t0, dp)
    r = _hop_kernel(t1, src3, dst3, zeros_d)
    out = _fin(r, t1, dp, b.reshape(1, D))
    return out[:N]


# speedup vs baseline: 2.0275x; 2.0275x over previous
import functools

import jax
import jax.numpy as jnp
from jax import lax
from jax.experimental import pallas as pl
from jax.experimental.pallas import tpu as pltpu
from jax.experimental.pallas import tpu_sc as plsc

N = 10000
E = 320000
IN_CH = 128
D = 64
NP = 10240
NCORES = 2
NSUB = 16
NTILES = NCORES * NSUB
CHUNK = 128
CPT = 80
EP = NTILES * CPT * CHUNK
STRIPE = NP // NSUB
DEGW = 16
RB = 1024

_mesh = plsc.VectorSubcoreMesh(core_axis_name="c", subcore_axis_name="s")
_SC_PARAMS = pltpu.CompilerParams(use_tc_tiling_on_sc=False)


@functools.partial(
    pl.kernel,
    out_type=jax.ShapeDtypeStruct((NCORES, NP, DEGW), jnp.float32),
    mesh=_mesh,
    scratch_types=[
        pltpu.VMEM((CPT, CHUNK), jnp.int32),
        pltpu.VMEM((CHUNK, DEGW), jnp.float32),
        pltpu.VMEM_SHARED((NP, DEGW), jnp.float32),
    ],
    compiler_params=_SC_PARAMS,
)
def _deg_kernel(dst_hbm, ones_hbm, zeros_hbm, out_hbm, didx, ones_v, acc):
    cid = lax.axis_index("c")
    sid = lax.axis_index("s")
    wid = sid * NCORES + cid
    pltpu.sync_copy(zeros_hbm.at[pl.ds(sid * STRIPE, STRIPE)],
                    acc.at[pl.ds(sid * STRIPE, STRIPE)])
    pltpu.sync_copy(dst_hbm.at[wid], didx)
    pltpu.sync_copy(ones_hbm, ones_v)
    plsc.subcore_barrier()

    @pl.loop(0, CPT)
    def _(j):
        pltpu.sync_copy(ones_v, acc.at[didx.at[j]], add=True)

    plsc.subcore_barrier()
    pltpu.sync_copy(acc.at[pl.ds(sid * STRIPE, STRIPE)],
                    out_hbm.at[cid, pl.ds(sid * STRIPE, STRIPE)])


@functools.partial(
    pl.kernel,
    out_type=jax.ShapeDtypeStruct((NCORES, NP, D), jnp.float32),
    mesh=_mesh,
    scratch_types=[
        pltpu.VMEM((CPT, CHUNK), jnp.int32),
        pltpu.VMEM((CPT, CHUNK), jnp.int32),
        pltpu.VMEM((CHUNK, D), jnp.float32),
        pltpu.VMEM_SHARED((NP, D), jnp.float32),
    ],
    compiler_params=_SC_PARAMS,
)
def _hop_kernel(t_hbm, src_hbm, dst_hbm, zeros_hbm, out_hbm,
                sidx, didx, rows_a, acc):
    cid = lax.axis_index("c")
    sid = lax.axis_index("s")
    wid = sid * NCORES + cid
    pltpu.sync_copy(zeros_hbm.at[pl.ds(sid * STRIPE, STRIPE)],
                    acc.at[pl.ds(sid * STRIPE, STRIPE)])
    pltpu.sync_copy(src_hbm.at[wid], sidx)
    pltpu.sync_copy(dst_hbm.at[wid], didx)
    plsc.subcore_barrier()

    @pl.loop(0, CPT)
    def _(j):
        pltpu.sync_copy(t_hbm.at[sidx.at[j]], rows_a)
        pltpu.sync_copy(rows_a, acc.at[didx.at[j]], add=True)

    plsc.subcore_barrier()
    pltpu.sync_copy(acc.at[pl.ds(sid * STRIPE, STRIPE)],
                    out_hbm.at[cid, pl.ds(sid * STRIPE, STRIPE)])


def _mm_body(x_ref, w_ref, o_ref):
    o_ref[...] = lax.dot_general(
        x_ref[...], w_ref[...], (((1,), (1,)), ((), ())),
        preferred_element_type=jnp.float32,
        precision=lax.Precision.HIGHEST)


_matmul = pl.pallas_call(
    _mm_body,
    grid=(NP // RB,),
    in_specs=[pl.BlockSpec((RB, IN_CH), lambda i: (i, 0)),
              pl.BlockSpec((D, IN_CH), lambda i: (0, 0))],
    out_specs=pl.BlockSpec((RB, D), lambda i: (i, 0)),
    out_shape=jax.ShapeDtypeStruct((NP, D), jnp.float32),
)


def _deg_of(dp_ref):
    return dp_ref[0, :, 0:1] + dp_ref[1, :, 0:1] + 1.0


def _scale_body(h_ref, dp_ref, o_ref):
    o_ref[...] = h_ref[...] * lax.rsqrt(_deg_of(dp_ref))


def _mid_body(q_ref, t_ref, dp_ref, o_ref):
    o_ref[...] = (q_ref[0] + q_ref[1] + t_ref[...]) / _deg_of(dp_ref)


def _fin_body(r_ref, t_ref, dp_ref, b_ref, o_ref):
    o_ref[...] = ((r_ref[0] + r_ref[1] + t_ref[...])
                  * lax.rsqrt(_deg_of(dp_ref)) + b_ref[...])


_T_SPEC = pl.BlockSpec((RB, D), lambda i: (i, 0))
_P_SPEC = pl.BlockSpec((NCORES, RB, D), lambda i: (0, i, 0))
_DP_SPEC = pl.BlockSpec((NCORES, RB, DEGW), lambda i: (0, i, 0))
_OUT_T = jax.ShapeDtypeStruct((NP, D), jnp.float32)

_scale = pl.pallas_call(
    _scale_body, grid=(NP // RB,),
    in_specs=[_T_SPEC, _DP_SPEC], out_specs=_T_SPEC, out_shape=_OUT_T)

_mid = pl.pallas_call(
    _mid_body, grid=(NP // RB,),
    in_specs=[_P_SPEC, _T_SPEC, _DP_SPEC], out_specs=_T_SPEC, out_shape=_OUT_T)

_fin = pl.pallas_call(
    _fin_body, grid=(NP // RB,),
    in_specs=[_P_SPEC, _T_SPEC, _DP_SPEC,
              pl.BlockSpec((1, D), lambda i: (0, 0))],
    out_specs=_T_SPEC, out_shape=_OUT_T)


def kernel(x, edge_index, W, b):
    src = edge_index[0].astype(jnp.int32)
    dst = edge_index[1].astype(jnp.int32)
    pad = EP - E
    pad_dst = N + jnp.arange(pad, dtype=jnp.int32) % (NP - N)
    src3 = jnp.concatenate([src, pad_dst]).reshape(NTILES, CPT, CHUNK)
    dst3 = jnp.concatenate([dst, pad_dst]).reshape(NTILES, CPT, CHUNK)
    xp = jnp.pad(x, ((0, NP - N), (0, 0)))
    zeros_d = jnp.zeros((NP, D), jnp.float32)
    zeros_g = jnp.zeros((NP, DEGW), jnp.float32)
    ones_g = jnp.ones((CHUNK, DEGW), jnp.float32)

    h0 = _matmul(xp, W)
    dp = _deg_kernel(dst3, ones_g, zeros_g)
    t0 = _scale(h0, dp)
    q = _hop_kernel(t0, src3, dst3, zeros_d)
    t1 = _mid(q, t0, dp)
    r = _hop_kernel(t1, src3, dst3, zeros_d)
    out = _fin(r, t1, dp, b.reshape(1, D))
    return out[:N]


# speedup vs baseline: 2.0953x; 1.0334x over previous
import functools

import jax
import jax.numpy as jnp
from jax import lax
from jax.experimental import pallas as pl
from jax.experimental.pallas import tpu as pltpu
from jax.experimental.pallas import tpu_sc as plsc

N = 10000
E = 320000
IN_CH = 128
D = 64
NP = 10240
NCORES = 2
NSUB = 16
NTILES = NCORES * NSUB
CHUNK = 128
CPT = 80
EP = NTILES * CPT * CHUNK
STRIPE = NP // NSUB
DEGW = 16
RB = 1024

_mesh = plsc.VectorSubcoreMesh(core_axis_name="c", subcore_axis_name="s")
_SC_PARAMS = pltpu.CompilerParams(use_tc_tiling_on_sc=False)


@functools.partial(
    pl.kernel,
    out_type=jax.ShapeDtypeStruct((NCORES, NP, DEGW), jnp.float32),
    mesh=_mesh,
    scratch_types=[
        pltpu.VMEM((CPT, CHUNK), jnp.int32),
        pltpu.VMEM((CHUNK, DEGW), jnp.float32),
        pltpu.VMEM_SHARED((NP, DEGW), jnp.float32),
    ],
    compiler_params=_SC_PARAMS,
)
def _deg_kernel(dst_hbm, ones_hbm, zeros_hbm, out_hbm, didx, ones_v, acc):
    cid = lax.axis_index("c")
    sid = lax.axis_index("s")
    wid = sid * NCORES + cid
    pltpu.sync_copy(zeros_hbm.at[pl.ds(sid * STRIPE, STRIPE)],
                    acc.at[pl.ds(sid * STRIPE, STRIPE)])
    pltpu.sync_copy(dst_hbm.at[wid], didx)
    pltpu.sync_copy(ones_hbm, ones_v)
    plsc.subcore_barrier()

    @pl.loop(0, CPT)
    def _(j):
        pltpu.sync_copy(ones_v, acc.at[didx.at[j]], add=True)

    plsc.subcore_barrier()
    pltpu.sync_copy(acc.at[pl.ds(sid * STRIPE, STRIPE)],
                    out_hbm.at[cid, pl.ds(sid * STRIPE, STRIPE)])


@functools.partial(
    pl.kernel,
    out_type=jax.ShapeDtypeStruct((NCORES, NP, D), jnp.float32),
    mesh=_mesh,
    scratch_types=[
        pltpu.VMEM((CPT, CHUNK), jnp.int32),
        pltpu.VMEM((CPT, CHUNK), jnp.int32),
        pltpu.VMEM((CHUNK, D), jnp.float32),
        pltpu.VMEM_SHARED((NP, D), jnp.float32),
        pltpu.VMEM_SHARED((NP, D), jnp.float32),
    ],
    compiler_params=_SC_PARAMS,
)
def _hop_kernel(t_hbm, src_hbm, dst_hbm, zeros_hbm, out_hbm,
                sidx, didx, rows_a, tbl, acc):
    cid = lax.axis_index("c")
    sid = lax.axis_index("s")
    wid = sid * NCORES + cid
    pltpu.sync_copy(t_hbm.at[pl.ds(sid * STRIPE, STRIPE)],
                    tbl.at[pl.ds(sid * STRIPE, STRIPE)])
    pltpu.sync_copy(zeros_hbm.at[pl.ds(sid * STRIPE, STRIPE)],
                    acc.at[pl.ds(sid * STRIPE, STRIPE)])
    pltpu.sync_copy(src_hbm.at[wid], sidx)
    pltpu.sync_copy(dst_hbm.at[wid], didx)
    plsc.subcore_barrier()

    @pl.loop(0, CPT)
    def _(j):
        pltpu.sync_copy(tbl.at[sidx.at[j]], rows_a)
        pltpu.sync_copy(rows_a, acc.at[didx.at[j]], add=True)

    plsc.subcore_barrier()
    pltpu.sync_copy(acc.at[pl.ds(sid * STRIPE, STRIPE)],
                    out_hbm.at[cid, pl.ds(sid * STRIPE, STRIPE)])


def _mm_body(x_ref, w_ref, o_ref):
    o_ref[...] = lax.dot_general(
        x_ref[...], w_ref[...], (((1,), (1,)), ((), ())),
        preferred_element_type=jnp.float32,
        precision=lax.Precision.HIGHEST)


_matmul = pl.pallas_call(
    _mm_body,
    grid=(NP // RB,),
    in_specs=[pl.BlockSpec((RB, IN_CH), lambda i: (i, 0)),
              pl.BlockSpec((D, IN_CH), lambda i: (0, 0))],
    out_specs=pl.BlockSpec((RB, D), lambda i: (i, 0)),
    out_shape=jax.ShapeDtypeStruct((NP, D), jnp.float32),
)


def _deg_of(dp_ref):
    return dp_ref[0, :, 0:1] + dp_ref[1, :, 0:1] + 1.0


def _scale_body(h_ref, dp_ref, o_ref):
    o_ref[...] = h_ref[...] * lax.rsqrt(_deg_of(dp_ref))


def _mid_body(q_ref, t_ref, dp_ref, o_ref):
    o_ref[...] = (q_ref[0] + q_ref[1] + t_ref[...]) / _deg_of(dp_ref)


def _fin_body(r_ref, t_ref, dp_ref, b_ref, o_ref):
    o_ref[...] = ((r_ref[0] + r_ref[1] + t_ref[...])
                  * lax.rsqrt(_deg_of(dp_ref)) + b_ref[...])


_T_SPEC = pl.BlockSpec((RB, D), lambda i: (i, 0))
_P_SPEC = pl.BlockSpec((NCORES, RB, D), lambda i: (0, i, 0))
_DP_SPEC = pl.BlockSpec((NCORES, RB, DEGW), lambda i: (0, i, 0))
_OUT_T = jax.ShapeDtypeStruct((NP, D), jnp.float32)

_scale = pl.pallas_call(
    _scale_body, grid=(NP // RB,),
    in_specs=[_T_SPEC, _DP_SPEC], out_specs=_T_SPEC, out_shape=_OUT_T)

_mid = pl.pallas_call(
    _mid_body, grid=(NP // RB,),
    in_specs=[_P_SPEC, _T_SPEC, _DP_SPEC], out_specs=_T_SPEC, out_shape=_OUT_T)

_fin = pl.pallas_call(
    _fin_body, grid=(NP // RB,),
    in_specs=[_P_SPEC, _T_SPEC, _DP_SPEC,
              pl.BlockSpec((1, D), lambda i: (0, 0))],
    out_specs=_T_SPEC, out_shape=_OUT_T)


def kernel(x, edge_index, W, b):
    src = edge_index[0].astype(jnp.int32)
    dst = edge_index[1].astype(jnp.int32)
    pad = EP - E
    pad_dst = N + jnp.arange(pad, dtype=jnp.int32) % (NP - N)
    src3 = jnp.concatenate([src, pad_dst]).reshape(NTILES, CPT, CHUNK)
    dst3 = jnp.concatenate([dst, pad_dst]).reshape(NTILES, CPT, CHUNK)
    xp = jnp.pad(x, ((0, NP - N), (0, 0)))
    zeros_d = jnp.zeros((NP, D), jnp.float32)
    zeros_g = jnp.zeros((NP, DEGW), jnp.float32)
    ones_g = jnp.ones((CHUNK, DEGW), jnp.float32)

    h0 = _matmul(xp, W)
    dp = _deg_kernel(dst3, ones_g, zeros_g)
    t0 = _scale(h0, dp)
    q = _hop_kernel(t0, src3, dst3, zeros_d)
    t1 = _mid(q, t0, dp)
    r = _hop_kernel(t1, src3, dst3, zeros_d)
    out = _fin(r, t1, dp, b.reshape(1, D))
    return out[:N]
